# baseline (speedup 1.0000x reference)
import functools

import jax
import jax.numpy as jnp
from jax import lax
from jax.experimental import pallas as pl
from jax.experimental.pallas import tpu as pltpu
from jax.experimental.pallas import tpu_sc as plsc

_PS = 16
_D = 768
_N = 576
_NP = 24

_SC_CORES = 2
_NW = 32
_CLS_ROWS = 8


def _sc_kernel(x5, y, table, B):
    mesh = plsc.VectorSubcoreMesh(core_axis_name="c", subcore_axis_name="s")
    tpw = (B * _NP) // _NW

    @functools.partial(
        pl.kernel,
        mesh=mesh,
        out_type=jax.ShapeDtypeStruct((B, 1 + _N, _D), jnp.float32),
        scratch_types=[
            pltpu.VMEM((_CLS_ROWS,), jnp.int32),
            pltpu.VMEM((_CLS_ROWS, _D), jnp.float32),
            pltpu.VMEM((2, 3, _PS, _NP * _PS), jnp.float32),
            pltpu.VMEM((4, _NP, _D), jnp.float32),
            pltpu.SemaphoreType.DMA,
            pltpu.SemaphoreType.DMA,
            pltpu.SemaphoreType.DMA,
            pltpu.SemaphoreType.DMA,
            pltpu.SemaphoreType.DMA,
            pltpu.SemaphoreType.DMA,
        ],
    )
    def body(x_hbm, y_hbm, table_hbm, out_hbm, idx_v, rows_v, buf, buf2,
             gsem0, gsem1, ssem0, ssem1, ssem2, ssem3):
        wid = lax.axis_index("s") * _SC_CORES + lax.axis_index("c")
        gsems = (gsem0, gsem1)
        ssems = (ssem0, ssem1, ssem2, ssem3)

        def bph(t):
            task_id = wid * tpw + t
            return task_id // _NP, task_id % _NP

        def gather(t, slot):
            b, ph = bph(t)
            return pltpu.make_async_copy(
                x_hbm.at[b, :, ph, :, :], buf.at[slot], gsems[slot])

        def relayout_scatter(t, gslot, sslot):
            b, ph = bph(t)
            for q in range(_NP):
                for c in range(3):
                    for i in range(_PS):
                        buf2[sslot, q, pl.ds(c * 256 + i * _PS, _PS)] = (
                            buf[gslot, c, i, pl.ds(q * _PS, _PS)])
            for q in range(_NP):
                pltpu.make_async_copy(
                    buf2.at[sslot, q],
                    out_hbm.at[b, 1 + ph * _NP + q, :],
                    ssems[sslot]).start()

        def drain_scatters(slot):
            pltpu.make_async_copy(
                out_hbm.at[0, pl.ds(8, _NP), :], buf2.at[slot],
                ssems[slot]).wait()

        def process(t, gslot, sslot, first):
            @pl.when(t + 1 < tpw)
            def _():
                gather(t + 1, 1 - gslot).start()
            gather(t, gslot).wait()
            @pl.when(jnp.logical_not(first))
            def _():
                drain_scatters(sslot)
            relayout_scatter(t, gslot, sslot)

        gather(0, 0).start()

        @pl.when(wid < B // _CLS_ROWS)
        def _():
            base = wid * _CLS_ROWS
            pltpu.sync_copy(y_hbm.at[pl.ds(base, _CLS_ROWS)], idx_v)
            pltpu.async_copy(table_hbm.at[idx_v], rows_v, ssem0).wait()
            for k in range(_CLS_ROWS):
                pltpu.sync_copy(rows_v.at[k], out_hbm.at[base + k, 0, :])

        def step(tt, carry):
            first = tt == 0
            for k in range(4):
                process(4 * tt + k, k % 2, k, first)
            return carry
        lax.fori_loop(0, tpw // 4, step, 0)

        for k in range(4):
            drain_scatters(k)

    return body(x5, y, table)


def kernel(x, y, label_bank):
    B, C, H, W = x.shape
    table = label_bank.reshape(label_bank.shape[0], _D)
    x5 = x.reshape(B, C, _NP, _PS, W)
    return _sc_kernel(x5, y, table, B)


# speedup vs baseline: 1.0864x; 1.0864x over previous
import functools

import jax
import jax.numpy as jnp
from jax import lax
from jax.experimental import pallas as pl
from jax.experimental.pallas import tpu as pltpu
from jax.experimental.pallas import tpu_sc as plsc

_PS = 16
_D = 768
_N = 576
_NP = 24

_SC_CORES = 2
_NW = 32
_CLS_ROWS = 8


def _sc_kernel(x5, y, table, B):
    mesh = plsc.VectorSubcoreMesh(core_axis_name="c", subcore_axis_name="s")
    tpw = (B * _NP) // _NW

    @functools.partial(
        pl.kernel,
        mesh=mesh,
        out_type=jax.ShapeDtypeStruct((B, 1 + _N, _D), jnp.float32),
        scratch_types=[
            pltpu.VMEM((_CLS_ROWS,), jnp.int32),
            pltpu.VMEM((_CLS_ROWS, _D), jnp.float32),
            pltpu.VMEM((2, 3, _PS, _NP * _PS), jnp.float32),
            pltpu.VMEM((2, _NP, _D), jnp.float32),
            pltpu.SemaphoreType.DMA,
            pltpu.SemaphoreType.DMA,
            pltpu.SemaphoreType.DMA,
            pltpu.SemaphoreType.DMA,
        ],
    )
    def body(x_hbm, y_hbm, table_hbm, out_hbm, idx_v, rows_v, buf, buf2,
             gsem0, gsem1, ssem0, ssem1):
        wid = lax.axis_index("s") * _SC_CORES + lax.axis_index("c")
        gsems = (gsem0, gsem1)
        ssems = (ssem0, ssem1)

        def bph(t):
            task_id = wid * tpw + t
            return task_id // _NP, task_id % _NP

        def gather(t, slot):
            b, ph = bph(t)
            return pltpu.make_async_copy(
                x_hbm.at[b, :, ph, :, :], buf.at[slot], gsems[slot])

        def relayout_scatter(t, gslot, sslot):
            del gslot
            slot = sslot
            b, ph = bph(t)
            for q in range(_NP):
                for c in range(3):
                    for i in range(_PS):
                        buf2[slot, q, pl.ds(c * 256 + i * _PS, _PS)] = (
                            buf[slot, c, i, pl.ds(q * _PS, _PS)])
            for q in range(_NP):
                pltpu.make_async_copy(
                    buf2.at[slot, q],
                    out_hbm.at[b, 1 + ph * _NP + q, :],
                    ssems[slot]).start()

        def drain_scatters(slot):
            pltpu.make_async_copy(
                out_hbm.at[0, pl.ds(8, _NP), :], buf2.at[slot],
                ssems[slot]).wait()

        def process(t, gslot, sslot, first):
            @pl.when(t + 1 < tpw)
            def _():
                gather(t + 1, 1 - gslot).start()
            gather(t, gslot).wait()
            @pl.when(jnp.logical_not(first))
            def _():
                drain_scatters(sslot)
            relayout_scatter(t, gslot, sslot)

        gather(0, 0).start()

        @pl.when(wid < B // _CLS_ROWS)
        def _():
            base = wid * _CLS_ROWS
            pltpu.sync_copy(y_hbm.at[pl.ds(base, _CLS_ROWS)], idx_v)
            pltpu.async_copy(table_hbm.at[idx_v], rows_v, ssem0).wait()
            for k in range(_CLS_ROWS):
                pltpu.sync_copy(rows_v.at[k], out_hbm.at[base + k, 0, :])

        def step(tt, carry):
            first = tt == 0
            process(2 * tt, 0, 0, first)
            process(2 * tt + 1, 1, 1, first)
            return carry
        lax.fori_loop(0, tpw // 2, step, 0)

        drain_scatters(0)
        drain_scatters(1)

    return body(x5, y, table)


def kernel(x, y, label_bank):
    B, C, H, W = x.shape
    table = label_bank.reshape(label_bank.shape[0], _D)
    x5 = x.reshape(B, C, _NP, _PS, W)
    return _sc_kernel(x5, y, table, B)
